# Initial kernel scaffold; baseline (speedup 1.0000x reference)
import jax
import jax.numpy as jnp
from jax.experimental import pallas as pl


def kernel(input, D):
    raise NotImplementedError("write your pallas kernel here")


# speedup vs baseline: 2.4890x; 2.4890x over previous
import jax
import jax.numpy as jnp
from jax import lax
from jax.experimental import pallas as pl
from jax.experimental.pallas import tpu as pltpu
from jax.experimental.pallas import tpu_sc as plsc

N_ROWS = 320000
HALF = 160000
D_MODEL = 128
NSEG = 10000
NSUB = 16
SEG_PAD = 10368
SCATTER_W = 80
CHUNK = 8 * SCATTER_W
NSCATTER = 8
ROWBUF = 160
NCHUNK = HALF // CHUNK
MAX_ITER = -(-NCHUNK // NSUB)
SEG_SLICE = SEG_PAD // NSUB


def _sc_body(x_hbm, d_hbm, za_hbm,
             sums_out,
             rowbuf, idxbuf, sums_sh):
    c = lax.axis_index("c")
    t = lax.axis_index("s")

    seg0 = pl.multiple_of(t * SEG_SLICE, SEG_SLICE)

    pltpu.sync_copy(za_hbm, rowbuf)
    for k3 in range(4):
        pltpu.sync_copy(rowbuf,
                        sums_sh.at[pl.ds(seg0 + k3 * ROWBUF, ROWBUF)])
    pltpu.sync_copy(rowbuf.at[pl.ds(0, 8)],
                    sums_sh.at[pl.ds(seg0 + 4 * ROWBUF, 8)])
    plsc.subcore_barrier()

    def chunk(i, carry):
        m = t + NSUB * i

        @pl.when(m < NCHUNK)
        def _do():
            g0 = pl.multiple_of(c * HALF + m * CHUNK, CHUNK)
            pltpu.sync_copy(
                d_hbm.at[pl.ds(pl.multiple_of(g0 // SCATTER_W, NSCATTER),
                               NSCATTER)],
                idxbuf)
            for s in range(CHUNK // ROWBUF):
                pltpu.sync_copy(
                    x_hbm.at[pl.ds(pl.multiple_of(g0 + s * ROWBUF, NSCATTER),
                                   ROWBUF)],
                    rowbuf)
                for jj in range(ROWBUF // SCATTER_W):
                    j = s * (ROWBUF // SCATTER_W) + jj
                    pltpu.sync_copy(rowbuf.at[pl.ds(jj * SCATTER_W, SCATTER_W)],
                                    sums_sh.at[idxbuf.at[j]], add=True)

        return carry

    lax.fori_loop(0, MAX_ITER, chunk, 0)
    plsc.subcore_barrier()

    out0 = pl.multiple_of(c * SEG_PAD + seg0, SEG_SLICE)
    for k2 in range(4):
        pltpu.sync_copy(sums_sh.at[pl.ds(seg0 + k2 * ROWBUF, ROWBUF)], rowbuf)
        pltpu.sync_copy(rowbuf,
                        sums_out.at[pl.ds(out0 + k2 * ROWBUF, ROWBUF)])
    pltpu.sync_copy(sums_sh.at[pl.ds(seg0 + 4 * ROWBUF, 8)],
                    rowbuf.at[pl.ds(0, 8)])
    pltpu.sync_copy(rowbuf.at[pl.ds(0, 8)],
                    sums_out.at[pl.ds(out0 + 4 * ROWBUF, 8)])


def _sc_segment_sums(x, d2):
    mesh = plsc.VectorSubcoreMesh(core_axis_name="c", subcore_axis_name="s",
                                  num_cores=2, num_subcores=16)
    k = pl.kernel(
        _sc_body,
        out_type=jax.ShapeDtypeStruct((2 * SEG_PAD, D_MODEL), jnp.float32),
        mesh=mesh,
        scratch_types=[
            pltpu.VMEM((ROWBUF, D_MODEL), jnp.float32),
            pltpu.VMEM((NSCATTER, SCATTER_W), jnp.int32),
            pltpu.VMEM_SHARED((SEG_PAD, D_MODEL), jnp.float32),
        ],
    )
    za = jnp.zeros((ROWBUF, D_MODEL), jnp.float32)
    return k(x, d2, za)


ROWBLK = 3456
NSTEP = SEG_PAD // ROWBLK


def _tc_body(s_ref, c_ref, o_ref, acc_ref):
    k = pl.program_id(0)

    @pl.when(k == 0)
    def _init():
        acc_ref[...] = jnp.zeros((4, D_MODEL), jnp.float32)

    s = s_ref[...]
    cnt = jnp.sum(c_ref[...], axis=1)[:, :, None]
    s = s / jnp.maximum(cnt, 1.0)
    mx = jnp.max(s, axis=-1, keepdims=True)
    e = jnp.exp(s - mx)
    z = jnp.sum(e, axis=-1, keepdims=True)
    p = e / z
    lp = (s - mx) - jnp.log(z)
    seg = k * ROWBLK + lax.broadcasted_iota(jnp.int32, (2, ROWBLK, 1), 1)
    live = (seg < NSEG).astype(jnp.float32)
    acc_ref[0:2, :] += jnp.sum(p * live, axis=1)
    acc_ref[2:4, :] += jnp.sum(lp * live, axis=1)

    @pl.when(k == NSTEP - 1)
    def _final():
        acc = acc_ref[...] * (1.0 / NSEG)
        mp, mn, Lp, Ln = acc[0:1], acc[1:2], acc[2:3], acc[3:4]
        lmp = jnp.log(mp)
        lmn = jnp.log(mn)
        kl1 = jnp.sum(mp * (lmp - Lp))
        kl2 = jnp.sum(mn * (lmn - Ln))
        kl3 = jnp.sum(mn * (lmn - lmp))
        kl4 = jnp.sum(mp * (lmp - lmn))
        o_ref[0, 0] = jnp.maximum(1.0 + kl1 + kl2 - 0.5 * (kl3 + kl4), 0.0)


def _tc_loss(sums, cnts):
    return pl.pallas_call(
        _tc_body,
        grid=(NSTEP,),
        in_specs=[
            pl.BlockSpec((2, ROWBLK, D_MODEL), lambda k: (0, k, 0)),
            pl.BlockSpec((2, 1, ROWBLK), lambda k: (0, 0, k)),
        ],
        out_specs=pl.BlockSpec((1, 1), lambda k: (0, 0),
                               memory_space=pltpu.SMEM),
        out_shape=jax.ShapeDtypeStruct((1, 1), jnp.float32),
        scratch_shapes=[pltpu.VMEM((4, D_MODEL), jnp.float32)],
    )(sums, cnts)


def kernel(input, D):
    d2 = D.reshape(N_ROWS // SCATTER_W, SCATTER_W)
    sums = _sc_segment_sums(input, d2)
    pD, nD = jnp.split(D, 2, axis=0)
    on = jnp.ones((HALF,), jnp.float32)
    cnts = jnp.stack([
        jax.ops.segment_sum(on, pD, num_segments=SEG_PAD),
        jax.ops.segment_sum(on, nD, num_segments=SEG_PAD)])
    loss = _tc_loss(sums.reshape(2, SEG_PAD, D_MODEL),
                    cnts.reshape(2, 1, SEG_PAD))
    return loss[0, 0]
